# baseline (speedup 1.0000x reference)
import functools

import jax
import jax.numpy as jnp
from jax import lax
from jax.experimental import pallas as pl
from jax.experimental.pallas import tpu as pltpu
from jax.experimental.pallas import tpu_sc as plsc

EMBED = 64
PADE = 128
NBUF = 4
NUM_WORKERS = 32
LANES = 16


@functools.lru_cache(maxsize=None)
def _make_gather(batch: int, seq: int):
    bpw = batch // NUM_WORKERS
    n_groups = seq // NBUF
    mesh = plsc.VectorSubcoreMesh(core_axis_name="c", subcore_axis_name="s")

    @functools.partial(
        pl.kernel,
        mesh=mesh,
        out_type=jax.ShapeDtypeStruct((seq, EMBED, batch), jnp.float32),
        scratch_types=[
            pltpu.VMEM((seq, bpw), jnp.int32),
            pltpu.VMEM((NBUF, bpw, PADE), jnp.float32),
            pltpu.VMEM((2, EMBED, bpw), jnp.float32),
            pltpu.SemaphoreType.DMA((NBUF,)),
            pltpu.SemaphoreType.DMA((2,)),
        ],
        compiler_params=pltpu.CompilerParams(
            use_tc_tiling_on_sc=True, needs_layout_passes=False
        ),
    )
    def gather_kernel(xt_hbm, table_hbm, out_hbm, idx_v, gbuf, obuf, gsem, osem):
        wid = lax.axis_index("s") * 2 + lax.axis_index("c")
        pltpu.sync_copy(xt_hbm.at[:, pl.ds(wid * bpw, bpw)], idx_v)

        def start(s, b):
            pltpu.async_copy(
                table_hbm.at[idx_v.at[s]], gbuf.at[b], gsem.at[b]
            )

        def out_desc(s, ob):
            return pltpu.make_async_copy(
                obuf.at[ob],
                out_hbm.at[s, pl.ds(0, EMBED), pl.ds(wid * bpw, bpw)],
                osem.at[ob],
            )

        def transpose_and_write(s, b, ob):
            pltpu.make_async_copy(
                table_hbm.at[idx_v.at[s]], gbuf.at[b], gsem.at[b]
            ).wait()

            def col(c, carry):
                iot = lax.iota(jnp.int32, LANES)
                for k in range(EMBED // LANES):
                    e_v = iot + (LANES * k)
                    c_v = jnp.full((LANES,), 0, jnp.int32) + c
                    v = gbuf[b, c, pl.ds(LANES * k, LANES)]
                    plsc.store_scatter(obuf.at[ob], [e_v, c_v], v)
                return carry

            lax.fori_loop(0, bpw, col, 0)
            out_desc(s, ob).start()

        for b in range(NBUF):
            start(b, b)
        for b in range(NBUF):
            if b >= 2:
                out_desc(b - 2, b % 2).wait()
            transpose_and_write(b, b, b % 2)
            start(b + NBUF, b)

        def group(g, carry):
            s0 = g * NBUF
            for b in range(NBUF):
                s = s0 + b
                out_desc(s - 2, b % 2).wait()
                transpose_and_write(s, b, b % 2)
                start(s + NBUF, b)
            return carry

        lax.fori_loop(1, n_groups - 1, group, 0)

        s0 = (n_groups - 1) * NBUF
        for b in range(NBUF):
            s = s0 + b
            out_desc(s - 2, b % 2).wait()
            transpose_and_write(s, b, b % 2)
        for b in range(2):
            out_desc(s0 + 2 + b, b % 2).wait()

    return gather_kernel


def kernel(x, table):
    batch, seq = x.shape
    table_pad = jnp.pad(table, ((0, 0), (0, PADE - EMBED)))
    out_t = _make_gather(batch, seq)(x.T, table_pad)
    return out_t.transpose(2, 0, 1)


# speedup vs baseline: 1.0292x; 1.0292x over previous
import functools

import jax
import jax.numpy as jnp
from jax import lax
from jax.experimental import pallas as pl
from jax.experimental.pallas import tpu as pltpu
from jax.experimental.pallas import tpu_sc as plsc

EMBED = 64
PADE = 128
NBUF = 4
NUM_WORKERS = 32
LANES = 16


@functools.lru_cache(maxsize=None)
def _make_gather(batch: int, seq: int):
    bpw = batch // NUM_WORKERS
    n_groups = seq // NBUF
    mesh = plsc.VectorSubcoreMesh(core_axis_name="c", subcore_axis_name="s")

    @functools.partial(
        pl.kernel,
        mesh=mesh,
        out_type=jax.ShapeDtypeStruct((seq, EMBED, batch), jnp.float32),
        scratch_types=[
            pltpu.VMEM((seq, bpw), jnp.int32),
            pltpu.VMEM((NBUF, bpw, PADE), jnp.float32),
            pltpu.VMEM((2, EMBED, bpw), jnp.float32),
            pltpu.SemaphoreType.DMA((NBUF,)),
            pltpu.SemaphoreType.DMA((2,)),
        ],
        compiler_params=pltpu.CompilerParams(
            use_tc_tiling_on_sc=True, needs_layout_passes=False
        ),
    )
    def gather_kernel(xt_hbm, table_hbm, out_hbm, idx_v, gbuf, obuf, gsem, osem):
        wid = lax.axis_index("s") * 2 + lax.axis_index("c")
        pltpu.sync_copy(xt_hbm.at[:, pl.ds(wid * bpw, bpw)], idx_v)

        def start(s, b):
            pltpu.async_copy(
                table_hbm.at[idx_v.at[s]], gbuf.at[b], gsem.at[b]
            )

        def out_desc(s, ob):
            return pltpu.make_async_copy(
                obuf.at[ob],
                out_hbm.at[s, pl.ds(0, EMBED), pl.ds(wid * bpw, bpw)],
                osem.at[ob],
            )

        def transpose_and_write(s, b, ob):
            pltpu.make_async_copy(
                table_hbm.at[idx_v.at[s]], gbuf.at[b], gsem.at[b]
            ).wait()

            def col(ci, carry):
                iot = lax.iota(jnp.int32, LANES)
                c0 = ci * 8
                vals = []
                for dc in range(8):
                    c = c0 + dc
                    c_v = jnp.full((LANES,), 0, jnp.int32) + c
                    for k in range(EMBED // LANES):
                        vals.append(
                            (iot + (LANES * k), c_v, gbuf[b, c, pl.ds(LANES * k, LANES)])
                        )
                for e_v, c_v, v in vals:
                    plsc.store_scatter(obuf.at[ob], [e_v, c_v], v)
                return carry

            lax.fori_loop(0, bpw // 8, col, 0)
            out_desc(s, ob).start()

        for b in range(NBUF):
            start(b, b)
        for b in range(NBUF):
            if b >= 2:
                out_desc(b - 2, b % 2).wait()
            transpose_and_write(b, b, b % 2)
            start(b + NBUF, b)

        def group(g, carry):
            s0 = g * NBUF
            for b in range(NBUF):
                s = s0 + b
                out_desc(s - 2, b % 2).wait()
                transpose_and_write(s, b, b % 2)
                start(s + NBUF, b)
            return carry

        lax.fori_loop(1, n_groups - 1, group, 0)

        s0 = (n_groups - 1) * NBUF
        for b in range(NBUF):
            s = s0 + b
            out_desc(s - 2, b % 2).wait()
            transpose_and_write(s, b, b % 2)
        for b in range(2):
            out_desc(s0 + 2 + b, b % 2).wait()

    return gather_kernel


def kernel(x, table):
    batch, seq = x.shape
    table_pad = jnp.pad(table, ((0, 0), (0, PADE - EMBED)))
    out_t = _make_gather(batch, seq)(x.T, table_pad)
    return out_t.transpose(2, 0, 1)


# speedup vs baseline: 1.3461x; 1.3079x over previous
import functools

import jax
import jax.numpy as jnp
from jax import lax
from jax.experimental import pallas as pl
from jax.experimental.pallas import tpu as pltpu
from jax.experimental.pallas import tpu_sc as plsc

EMBED = 64
PADE = 128
NBUF = 4
NUM_WORKERS = 32
LANES = 16


@functools.lru_cache(maxsize=None)
def _make_gather(batch: int, seq: int):
    bpw = batch // NUM_WORKERS
    n_groups = seq // NBUF
    mesh = plsc.VectorSubcoreMesh(core_axis_name="c", subcore_axis_name="s")

    @functools.partial(
        pl.kernel,
        mesh=mesh,
        out_type=jax.ShapeDtypeStruct((seq, EMBED, batch), jnp.float32),
        scratch_types=[
            pltpu.VMEM((seq, bpw), jnp.int32),
            pltpu.VMEM((NBUF, bpw, PADE), jnp.float32),
            pltpu.VMEM((2, EMBED, bpw), jnp.float32),
            pltpu.SemaphoreType.DMA((NBUF,)),
            pltpu.SemaphoreType.DMA((2,)),
        ],
        compiler_params=pltpu.CompilerParams(
            use_tc_tiling_on_sc=True, needs_layout_passes=False
        ),
    )
    def gather_kernel(xt_hbm, table_hbm, out_hbm, idx_v, gbuf, obuf, gsem, osem):
        wid = lax.axis_index("s") * 2 + lax.axis_index("c")
        pltpu.sync_copy(xt_hbm.at[:, pl.ds(wid * bpw, bpw)], idx_v)

        def start(s, b):
            pltpu.async_copy(
                table_hbm.at[idx_v.at[s]], gbuf.at[b], gsem.at[b]
            )

        def out_desc(s, ob):
            return pltpu.make_async_copy(
                obuf.at[ob],
                out_hbm.at[s, pl.ds(0, EMBED), pl.ds(wid * bpw, bpw)],
                osem.at[ob],
            )

        def transpose_and_write(s, b, ob):
            pltpu.make_async_copy(
                table_hbm.at[idx_v.at[s]], gbuf.at[b], gsem.at[b]
            ).wait()

            def cblk(ci, carry):
                iot = lax.iota(jnp.int32, LANES)
                c_v = ci * LANES + iot
                for eb in range(EMBED // LANES):
                    for d in range(LANES):
                        e_v = (LANES * eb) + jnp.bitwise_and(iot + d, LANES - 1)
                        v = plsc.load_gather(gbuf.at[b], [c_v, e_v])
                        plsc.store_scatter(obuf.at[ob], [e_v, c_v], v)
                return carry

            lax.fori_loop(0, bpw // LANES, cblk, 0)
            out_desc(s, ob).start()

        for b in range(NBUF):
            start(b, b)
        for b in range(NBUF):
            if b >= 2:
                out_desc(b - 2, b % 2).wait()
            transpose_and_write(b, b, b % 2)
            start(b + NBUF, b)

        def group(g, carry):
            s0 = g * NBUF
            for b in range(NBUF):
                s = s0 + b
                out_desc(s - 2, b % 2).wait()
                transpose_and_write(s, b, b % 2)
                start(s + NBUF, b)
            return carry

        lax.fori_loop(1, n_groups - 1, group, 0)

        s0 = (n_groups - 1) * NBUF
        for b in range(NBUF):
            s = s0 + b
            out_desc(s - 2, b % 2).wait()
            transpose_and_write(s, b, b % 2)
        for b in range(2):
            out_desc(s0 + 2 + b, b % 2).wait()

    return gather_kernel


def kernel(x, table):
    batch, seq = x.shape
    table_pad = jnp.pad(table, ((0, 0), (0, PADE - EMBED)))
    out_t = _make_gather(batch, seq)(x.T, table_pad)
    return out_t.transpose(2, 0, 1)


# speedup vs baseline: 1.6333x; 1.2134x over previous
import functools

import jax
import jax.numpy as jnp
from jax import lax
from jax.experimental import pallas as pl
from jax.experimental.pallas import tpu as pltpu
from jax.experimental.pallas import tpu_sc as plsc

EMBED = 64
PADE = 128
NBUF = 4
NUM_WORKERS = 32
CL0, CL1 = 104, 96


@functools.lru_cache(maxsize=None)
def _make_gather(batch: int, seq: int):
    assert seq == CL0 + CL1
    rows_per_worker = batch // NUM_WORKERS
    n_steps = rows_per_worker * 2
    n_groups = n_steps // NBUF
    mesh = plsc.VectorSubcoreMesh(core_axis_name="c", subcore_axis_name="s")

    def step_params(s):
        return s // 2, (s % 2) * CL0, CL0 if s % 2 == 0 else CL1

    @functools.partial(
        pl.kernel,
        mesh=mesh,
        out_type=jax.ShapeDtypeStruct((batch, seq, PADE), jnp.float32),
        scratch_types=[
            pltpu.VMEM((rows_per_worker, seq), jnp.int32),
            pltpu.VMEM((NBUF, CL0, PADE), jnp.float32),
            pltpu.SemaphoreType.DMA((NBUF,)),
        ],
        compiler_params=pltpu.CompilerParams(use_tc_tiling_on_sc=False),
    )
    def gather_kernel(idx_hbm, table_hbm, out_hbm, idx_v, rows_v, gsem):
        wid = lax.axis_index("s") * 2 + lax.axis_index("c")
        row0 = wid * rows_per_worker
        pltpu.sync_copy(idx_hbm.at[pl.ds(row0, rows_per_worker)], idx_v)

        def start(s_dyn, b, s0, cl):
            r = s_dyn // 2
            pltpu.async_copy(
                table_hbm.at[idx_v.at[r, pl.ds(s0, cl)]],
                rows_v.at[b, pl.ds(0, cl)],
                gsem.at[b],
            )

        def finish(s_dyn, b, s0, cl):
            r = s_dyn // 2
            pltpu.make_async_copy(
                table_hbm.at[idx_v.at[r, pl.ds(s0, cl)]],
                rows_v.at[b, pl.ds(0, cl)],
                gsem.at[b],
            ).wait()
            pltpu.sync_copy(
                rows_v.at[b, pl.ds(0, cl), pl.ds(0, EMBED)],
                out_hbm.at[row0 + r, pl.ds(s0, cl), pl.ds(0, EMBED)],
            )

        for b in range(NBUF):
            _, s0, cl = step_params(b)
            start(b, b, s0, cl)

        def group(g, carry):
            j0 = g * NBUF
            for b in range(NBUF):
                _, s0, cl = step_params(b)
                finish(j0 + b, b, s0, cl)
                start(j0 + b + NBUF, b, s0, cl)
            return carry

        lax.fori_loop(0, n_groups - 1, group, 0)

        j0 = (n_groups - 1) * NBUF
        for b in range(NBUF):
            _, s0, cl = step_params(b)
            finish(j0 + b, b, s0, cl)

    return gather_kernel


def kernel(x, table):
    batch, seq = x.shape
    table_pad = jnp.pad(table, ((0, 0), (0, PADE - EMBED)))
    out_pad = _make_gather(batch, seq)(x, table_pad)
    return out_pad[:, :, :EMBED]
